# baseline (speedup 1.0000x reference)
import functools

import jax
import jax.numpy as jnp
from jax import lax
from jax.experimental import pallas as pl
from jax.experimental.pallas import tpu as pltpu
from jax.experimental.pallas import tpu_sc as plsc

N_NODES = 10000
N_EDGES = 320000
D = 128
NUM_GRAPHS = 512

NC, NS = 2, 16
NW = NC * NS
CHUNK = 128
NCHUNK = 2560
TILE_CHUNKS = NCHUNK // NW
N_PAD_EDGES = NCHUNK * CHUNK - N_EDGES
ACC_ROWS = N_NODES + 8
STRIPE = 624
TAIL = N_NODES - NS * STRIPE
DEGW = 16

ROW_BLK = 2000
GRID = N_NODES // ROW_BLK


def _make_sc_scatter():
  mesh = plsc.VectorSubcoreMesh(core_axis_name="c", subcore_axis_name="s")
  out_type = [jax.ShapeDtypeStruct((NC, N_NODES, D), jnp.float32)]
  scratch = [
      pltpu.VMEM((4, CHUNK), jnp.int32),
      pltpu.VMEM((4, CHUNK), jnp.int32),
      pltpu.VMEM((2, CHUNK, D), jnp.float32),
      pltpu.VMEM((24, D), jnp.float32),
      pltpu.VMEM_SHARED((ACC_ROWS, D), jnp.float32),
      pltpu.SemaphoreType.DMA,
      pltpu.SemaphoreType.DMA,
      pltpu.SemaphoreType.DMA,
      pltpu.SemaphoreType.DMA,
      pltpu.SemaphoreType.DMA,
      pltpu.SemaphoreType.DMA,
      pltpu.SemaphoreType.DMA,
      pltpu.SemaphoreType.DMA,
  ]

  def body(src_hbm, dst_hbm, y_hbm, out_hbm, sidx, didx, rows, zbuf, acc_sh,
           rsem0, rsem1, ssem0, ssem1, isem0, isem1, isem2, isem3):
    c = lax.axis_index("c")
    s = lax.axis_index("s")
    w = c * NS + s
    rsem = [rsem0, rsem1]
    ssem = [ssem0, ssem1]
    isem = [isem0, isem1, isem2, isem3]

    def iload(ch, j):
      pltpu.async_copy(src_hbm.at[pl.ds(ch * CHUNK, CHUNK)], sidx.at[j],
                       isem[j])
      pltpu.async_copy(dst_hbm.at[pl.ds(ch * CHUNK, CHUNK)], didx.at[j],
                       isem[j])

    def iwait(j):
      pltpu.make_async_copy(src_hbm.at[pl.ds(0, CHUNK)], sidx.at[j],
                            isem[j]).wait()
      pltpu.make_async_copy(dst_hbm.at[pl.ds(0, CHUNK)], didx.at[j],
                            isem[j]).wait()

    def gstart(p, j):
      pltpu.async_copy(y_hbm.at[sidx.at[j]], rows.at[p], rsem[p])

    def gwait(p):
      pltpu.make_async_copy(y_hbm.at[sidx.at[0]], rows.at[p],
                            rsem[p]).wait()

    def sstart(p, j):
      pltpu.async_copy(rows.at[p], acc_sh.at[didx.at[j]], ssem[p], add=True)

    def swait(p):
      pltpu.make_async_copy(rows.at[p], acc_sh.at[pl.ds(0, CHUNK)],
                            ssem[p]).wait()

    for j in range(4):
      iload(j * NW + w, j)

    zv = jnp.zeros((16,), jnp.float32)
    for j in range(24):
      for k in range(D // 16):
        zbuf[j, pl.ds(k * 16, 16)] = zv
    zd = [pltpu.async_copy(zbuf, acc_sh.at[pl.ds(s * STRIPE + r * 24, 24)],
                           rsem0)
          for r in range(26)]

    @pl.when(s == NS - 1)
    def _():
      pltpu.sync_copy(zbuf.at[pl.ds(0, TAIL)],
                      acc_sh.at[pl.ds(NS * STRIPE, TAIL)])

    for d in zd:
      d.wait()

    iwait(0)
    gstart(0, 0)
    iwait(1)
    gstart(1, 1)
    plsc.subcore_barrier()

    def quad(i, carry):
      c0 = i * 4
      for x in range(4):
        p, j = x % 2, x
        gwait(p)
        sstart(p, j)
        swait(p)

        @pl.when(c0 + x + 2 < TILE_CHUNKS)
        def _(x=x, p=p):
          iwait((x + 2) % 4)
          gstart(p, (x + 2) % 4)

        @pl.when(c0 + x + 4 < TILE_CHUNKS)
        def _(x=x, j=j, c0=c0):
          iload((c0 + x + 4) * NW + w, j)

      return carry

    lax.fori_loop(0, TILE_CHUNKS // 4, quad, 0)
    plsc.subcore_barrier()

    pltpu.sync_copy(acc_sh.at[pl.ds(s * STRIPE, STRIPE)],
                    out_hbm.at[c, pl.ds(s * STRIPE, STRIPE)])

    @pl.when(s == NS - 1)
    def _():
      pltpu.sync_copy(acc_sh.at[pl.ds(NS * STRIPE, TAIL)],
                      out_hbm.at[c, pl.ds(NS * STRIPE, TAIL)])

  return pl.kernel(body, out_type=out_type, mesh=mesh, scratch_types=scratch)


_sc_scatter = _make_sc_scatter()


EDGE_BLK = 8000
EDGE_GRID = N_EDGES // EDGE_BLK
HI = (N_NODES + D - 1) // D


def _deg_body(dst_ref, inv_ref, acc):
  i = pl.program_id(0)

  @pl.when(i == 0)
  def _():
    acc[...] = jnp.zeros_like(acc)

  db = dst_ref[0]
  hi = db // D
  lo = db - hi * D
  ohi = (lax.broadcasted_iota(jnp.int32, (HI, EDGE_BLK), 0)
         == hi).astype(jnp.bfloat16)
  olo = (lax.broadcasted_iota(jnp.int32, (D, EDGE_BLK), 0)
         == lo).astype(jnp.bfloat16)
  acc[...] += lax.dot_general(ohi, olo, (((1,), (1,)), ((), ())),
                              preferred_element_type=jnp.float32)

  @pl.when(i == EDGE_GRID - 1)
  def _():
    inv_ref[...] = 1.0 / jnp.maximum(acc[...], 1.0)


def _invdeg(dst):
  out = pl.pallas_call(
      _deg_body,
      grid=(EDGE_GRID,),
      in_specs=[pl.BlockSpec((1, 1, EDGE_BLK), lambda i: (i, 0, 0))],
      out_specs=pl.BlockSpec((HI, D), lambda i: (0, 0)),
      out_shape=jax.ShapeDtypeStruct((HI, D), jnp.float32),
      scratch_shapes=[pltpu.VMEM((HI, D), jnp.float32)],
  )(dst.reshape(EDGE_GRID, 1, EDGE_BLK))
  return out.reshape(-1)[:N_NODES].reshape(N_NODES, 1)


def _layer_body(x_ref, sp_ref, inv_ref, ws_ref, wn_ref, b_ref, h_ref):
  agg = (sp_ref[0] + sp_ref[1]) * inv_ref[...]
  h_ref[...] = jnp.maximum(
      jnp.dot(x_ref[...], ws_ref[...], preferred_element_type=jnp.float32)
      + jnp.dot(agg, wn_ref[...], preferred_element_type=jnp.float32)
      + b_ref[...], 0.0)


def _layer(x, sp, invd, ws, wn, b):
  return pl.pallas_call(
      _layer_body,
      grid=(GRID,),
      in_specs=[
          pl.BlockSpec((ROW_BLK, D), lambda i: (i, 0)),
          pl.BlockSpec((NC, ROW_BLK, D), lambda i: (0, i, 0)),
          pl.BlockSpec((ROW_BLK, 1), lambda i: (i, 0)),
          pl.BlockSpec((D, D), lambda i: (0, 0)),
          pl.BlockSpec((D, D), lambda i: (0, 0)),
          pl.BlockSpec((1, D), lambda i: (0, 0)),
      ],
      out_specs=pl.BlockSpec((ROW_BLK, D), lambda i: (i, 0)),
      out_shape=jax.ShapeDtypeStruct((N_NODES, D), jnp.float32),
  )(x, sp, invd, ws, wn, b.reshape(1, D))


def _readout_body(h1_ref, sp_ref, inv_ref, ws_ref, wn_ref, b1_ref, b_ref,
                  wr0_ref, br0_ref, wr1_ref, br1_ref, wr2_ref, br2_ref,
                  out_ref, pooled, cnt):
  i = pl.program_id(0)

  @pl.when(i == 0)
  def _():
    pooled[...] = jnp.zeros_like(pooled)
    cnt[...] = jnp.zeros_like(cnt)

  agg = (sp_ref[0] + sp_ref[1]) * inv_ref[...]
  h = jnp.maximum(
      jnp.dot(h1_ref[...], ws_ref[...], preferred_element_type=jnp.float32)
      + jnp.dot(agg, wn_ref[...], preferred_element_type=jnp.float32)
      + b1_ref[...], 0.0)
  bb = b_ref[0]
  pt = (lax.broadcasted_iota(jnp.int32, (NUM_GRAPHS, ROW_BLK), 0)
        == bb).astype(jnp.float32)
  pooled[...] += jnp.dot(pt, h, preferred_element_type=jnp.float32,
                         precision=lax.Precision.HIGHEST)
  cnt[...] += jnp.sum(pt, axis=1, keepdims=True)

  @pl.when(i == GRID - 1)
  def _():
    pm = pooled[...] / jnp.maximum(cnt[...], 1.0)
    r = jnp.maximum(
        jnp.dot(pm, wr0_ref[...], preferred_element_type=jnp.float32)
        + br0_ref[...], 0.0)
    r = jnp.maximum(
        jnp.dot(r, wr1_ref[...], preferred_element_type=jnp.float32)
        + br1_ref[...], 0.0)
    out_ref[...] = jnp.dot(
        r, wr2_ref[...], preferred_element_type=jnp.float32) + br2_ref[...]


def _readout(h1, sp, invd, batch, ws, wn, b1, wr0, br0, wr1, br1, wr2, br2):
  hd = wr1.shape[1]
  return pl.pallas_call(
      _readout_body,
      grid=(GRID,),
      in_specs=[
          pl.BlockSpec((ROW_BLK, D), lambda i: (i, 0)),
          pl.BlockSpec((NC, ROW_BLK, D), lambda i: (0, i, 0)),
          pl.BlockSpec((ROW_BLK, 1), lambda i: (i, 0)),
          pl.BlockSpec((D, D), lambda i: (0, 0)),
          pl.BlockSpec((D, D), lambda i: (0, 0)),
          pl.BlockSpec((1, D), lambda i: (0, 0)),
          pl.BlockSpec((1, 1, ROW_BLK), lambda i: (i, 0, 0)),
          pl.BlockSpec((D, D), lambda i: (0, 0)),
          pl.BlockSpec((1, D), lambda i: (0, 0)),
          pl.BlockSpec((D, hd), lambda i: (0, 0)),
          pl.BlockSpec((1, hd), lambda i: (0, 0)),
          pl.BlockSpec((hd, 1), lambda i: (0, 0)),
          pl.BlockSpec((1, 1), lambda i: (0, 0)),
      ],
      out_specs=pl.BlockSpec((NUM_GRAPHS, 1), lambda i: (0, 0)),
      out_shape=jax.ShapeDtypeStruct((NUM_GRAPHS, 1), jnp.float32),
      scratch_shapes=[
          pltpu.VMEM((NUM_GRAPHS, D), jnp.float32),
          pltpu.VMEM((NUM_GRAPHS, 1), jnp.float32),
      ],
  )(h1, sp, invd, ws, wn, b1.reshape(1, D), batch.reshape(GRID, 1, ROW_BLK),
    wr0, br0.reshape(1, D), wr1, br1.reshape(1, hd), wr2, br2.reshape(1, 1))


@jax.jit
def kernel(x, edge_index, batch, W_self0, W_neigh0, b0, W_self1, W_neigh1, b1,
           Wr0, br0, Wr1, br1, Wr2, br2):
  src = edge_index[0].astype(jnp.int32)
  dst = edge_index[1].astype(jnp.int32)
  batch = batch.astype(jnp.int32)

  invd = _invdeg(dst)
  pad = jnp.arange(N_PAD_EDGES, dtype=jnp.int32) % 8
  srcp = jnp.concatenate([src, pad])
  dstp = jnp.concatenate([dst, N_NODES + pad])
  (s0p,) = _sc_scatter(srcp, dstp, x)
  h1 = _layer(x, s0p, invd, W_self0, W_neigh0, b0)
  (s1p,) = _sc_scatter(srcp, dstp, h1)
  return _readout(h1, s1p, invd, batch, W_self1, W_neigh1, b1,
                  Wr0, br0, Wr1, br1, Wr2, br2)


# speedup vs baseline: 1.0552x; 1.0552x over previous
import functools

import jax
import jax.numpy as jnp
from jax import lax
from jax.experimental import pallas as pl
from jax.experimental.pallas import tpu as pltpu
from jax.experimental.pallas import tpu_sc as plsc

N_NODES = 10000
N_EDGES = 320000
D = 128
NUM_GRAPHS = 512

NC, NS = 2, 16
NW = NC * NS
CHUNK = 128
NCHUNK = 2560
TILE_CHUNKS = NCHUNK // NW
N_PAD_EDGES = NCHUNK * CHUNK - N_EDGES
ACC_ROWS = N_NODES + 8
STRIPE = 624
TAIL = N_NODES - NS * STRIPE
DEGW = 16

ROW_BLK = 2000
GRID = N_NODES // ROW_BLK


def _make_sc_scatter():
  mesh = plsc.VectorSubcoreMesh(core_axis_name="c", subcore_axis_name="s")
  out_type = [jax.ShapeDtypeStruct((NC, N_NODES, D), jnp.float32)]
  scratch = [
      pltpu.VMEM((4, CHUNK), jnp.int32),
      pltpu.VMEM((4, CHUNK), jnp.int32),
      pltpu.VMEM((2, CHUNK, D), jnp.float32),
      pltpu.VMEM((24, D), jnp.float32),
      pltpu.VMEM_SHARED((ACC_ROWS, D), jnp.float32),
      pltpu.SemaphoreType.DMA,
      pltpu.SemaphoreType.DMA,
      pltpu.SemaphoreType.DMA,
      pltpu.SemaphoreType.DMA,
      pltpu.SemaphoreType.DMA,
      pltpu.SemaphoreType.DMA,
      pltpu.SemaphoreType.DMA,
      pltpu.SemaphoreType.DMA,
  ]

  def body(src_hbm, dst_hbm, y_hbm, out_hbm, sidx, didx, rows, zbuf, acc_sh,
           rsem0, rsem1, ssem0, ssem1, isem0, isem1, isem2, isem3):
    c = lax.axis_index("c")
    s = lax.axis_index("s")
    w = c * NS + s
    rsem = [rsem0, rsem1]
    ssem = [ssem0, ssem1]
    isem = [isem0, isem1, isem2, isem3]

    def iload(ch, j):
      pltpu.async_copy(src_hbm.at[pl.ds(ch * CHUNK, CHUNK)], sidx.at[j],
                       isem[j])
      pltpu.async_copy(dst_hbm.at[pl.ds(ch * CHUNK, CHUNK)], didx.at[j],
                       isem[j])

    def iwait(j):
      pltpu.make_async_copy(src_hbm.at[pl.ds(0, CHUNK)], sidx.at[j],
                            isem[j]).wait()
      pltpu.make_async_copy(dst_hbm.at[pl.ds(0, CHUNK)], didx.at[j],
                            isem[j]).wait()

    def gstart(p, j):
      pltpu.async_copy(y_hbm.at[sidx.at[j]], rows.at[p], rsem[p])

    def gwait(p):
      pltpu.make_async_copy(y_hbm.at[sidx.at[0]], rows.at[p],
                            rsem[p]).wait()

    def sstart(p, j):
      pltpu.async_copy(rows.at[p], acc_sh.at[didx.at[j]], ssem[p], add=True)

    def swait(p):
      pltpu.make_async_copy(rows.at[p], acc_sh.at[pl.ds(0, CHUNK)],
                            ssem[p]).wait()

    base = w * TILE_CHUNKS
    for j in range(4):
      iload(base + j, j)

    zv = jnp.zeros((16,), jnp.float32)
    for j in range(24):
      for k in range(D // 16):
        zbuf[j, pl.ds(k * 16, 16)] = zv
    zd = [pltpu.async_copy(zbuf, acc_sh.at[pl.ds(s * STRIPE + r * 24, 24)],
                           rsem0)
          for r in range(26)]

    @pl.when(s == NS - 1)
    def _():
      pltpu.sync_copy(zbuf.at[pl.ds(0, TAIL)],
                      acc_sh.at[pl.ds(NS * STRIPE, TAIL)])

    for d in zd:
      d.wait()

    iwait(0)
    gstart(0, 0)
    iwait(1)
    gstart(1, 1)
    plsc.subcore_barrier()

    def quad(i, carry):
      c0 = i * 4
      for x in range(4):
        p, j = x % 2, x
        gwait(p)
        sstart(p, j)
        swait(p)

        @pl.when(c0 + x + 2 < TILE_CHUNKS)
        def _(x=x, p=p):
          iwait((x + 2) % 4)
          gstart(p, (x + 2) % 4)

        @pl.when(c0 + x + 4 < TILE_CHUNKS)
        def _(x=x, j=j, c0=c0):
          iload(base + c0 + x + 4, j)

      return carry

    lax.fori_loop(0, TILE_CHUNKS // 4, quad, 0)
    plsc.subcore_barrier()

    pltpu.sync_copy(acc_sh.at[pl.ds(s * STRIPE, STRIPE)],
                    out_hbm.at[c, pl.ds(s * STRIPE, STRIPE)])

    @pl.when(s == NS - 1)
    def _():
      pltpu.sync_copy(acc_sh.at[pl.ds(NS * STRIPE, TAIL)],
                      out_hbm.at[c, pl.ds(NS * STRIPE, TAIL)])

  return pl.kernel(body, out_type=out_type, mesh=mesh, scratch_types=scratch)


_sc_scatter = _make_sc_scatter()


EDGE_BLK = 8000
EDGE_GRID = N_EDGES // EDGE_BLK
HI = (N_NODES + D - 1) // D


def _deg_body(dst_ref, inv_ref, acc):
  i = pl.program_id(0)

  @pl.when(i == 0)
  def _():
    acc[...] = jnp.zeros_like(acc)

  db = dst_ref[0]
  hi = db // D
  lo = db - hi * D
  ohi = (lax.broadcasted_iota(jnp.int32, (HI, EDGE_BLK), 0)
         == hi).astype(jnp.bfloat16)
  olo = (lax.broadcasted_iota(jnp.int32, (D, EDGE_BLK), 0)
         == lo).astype(jnp.bfloat16)
  acc[...] += lax.dot_general(ohi, olo, (((1,), (1,)), ((), ())),
                              preferred_element_type=jnp.float32)

  @pl.when(i == EDGE_GRID - 1)
  def _():
    inv_ref[...] = 1.0 / jnp.maximum(acc[...], 1.0)


def _invdeg(dst):
  out = pl.pallas_call(
      _deg_body,
      grid=(EDGE_GRID,),
      in_specs=[pl.BlockSpec((1, 1, EDGE_BLK), lambda i: (i, 0, 0))],
      out_specs=pl.BlockSpec((HI, D), lambda i: (0, 0)),
      out_shape=jax.ShapeDtypeStruct((HI, D), jnp.float32),
      scratch_shapes=[pltpu.VMEM((HI, D), jnp.float32)],
  )(dst.reshape(EDGE_GRID, 1, EDGE_BLK))
  return out.reshape(-1)[:N_NODES].reshape(N_NODES, 1)


def _layer_body(x_ref, sp_ref, inv_ref, ws_ref, wn_ref, b_ref, h_ref):
  agg = (sp_ref[0] + sp_ref[1]) * inv_ref[...]
  h_ref[...] = jnp.maximum(
      jnp.dot(x_ref[...], ws_ref[...], preferred_element_type=jnp.float32)
      + jnp.dot(agg, wn_ref[...], preferred_element_type=jnp.float32)
      + b_ref[...], 0.0)


def _layer(x, sp, invd, ws, wn, b):
  return pl.pallas_call(
      _layer_body,
      grid=(GRID,),
      in_specs=[
          pl.BlockSpec((ROW_BLK, D), lambda i: (i, 0)),
          pl.BlockSpec((NC, ROW_BLK, D), lambda i: (0, i, 0)),
          pl.BlockSpec((ROW_BLK, 1), lambda i: (i, 0)),
          pl.BlockSpec((D, D), lambda i: (0, 0)),
          pl.BlockSpec((D, D), lambda i: (0, 0)),
          pl.BlockSpec((1, D), lambda i: (0, 0)),
      ],
      out_specs=pl.BlockSpec((ROW_BLK, D), lambda i: (i, 0)),
      out_shape=jax.ShapeDtypeStruct((N_NODES, D), jnp.float32),
  )(x, sp, invd, ws, wn, b.reshape(1, D))


def _readout_body(h1_ref, sp_ref, inv_ref, ws_ref, wn_ref, b1_ref, b_ref,
                  wr0_ref, br0_ref, wr1_ref, br1_ref, wr2_ref, br2_ref,
                  out_ref, pooled, cnt):
  i = pl.program_id(0)

  @pl.when(i == 0)
  def _():
    pooled[...] = jnp.zeros_like(pooled)
    cnt[...] = jnp.zeros_like(cnt)

  agg = (sp_ref[0] + sp_ref[1]) * inv_ref[...]
  h = jnp.maximum(
      jnp.dot(h1_ref[...], ws_ref[...], preferred_element_type=jnp.float32)
      + jnp.dot(agg, wn_ref[...], preferred_element_type=jnp.float32)
      + b1_ref[...], 0.0)
  bb = b_ref[0]
  pt = (lax.broadcasted_iota(jnp.int32, (NUM_GRAPHS, ROW_BLK), 0)
        == bb).astype(jnp.float32)
  pooled[...] += jnp.dot(pt, h, preferred_element_type=jnp.float32,
                         precision=lax.Precision.HIGHEST)
  cnt[...] += jnp.sum(pt, axis=1, keepdims=True)

  @pl.when(i == GRID - 1)
  def _():
    pm = pooled[...] / jnp.maximum(cnt[...], 1.0)
    r = jnp.maximum(
        jnp.dot(pm, wr0_ref[...], preferred_element_type=jnp.float32)
        + br0_ref[...], 0.0)
    r = jnp.maximum(
        jnp.dot(r, wr1_ref[...], preferred_element_type=jnp.float32)
        + br1_ref[...], 0.0)
    out_ref[...] = jnp.dot(
        r, wr2_ref[...], preferred_element_type=jnp.float32) + br2_ref[...]


def _readout(h1, sp, invd, batch, ws, wn, b1, wr0, br0, wr1, br1, wr2, br2):
  hd = wr1.shape[1]
  return pl.pallas_call(
      _readout_body,
      grid=(GRID,),
      in_specs=[
          pl.BlockSpec((ROW_BLK, D), lambda i: (i, 0)),
          pl.BlockSpec((NC, ROW_BLK, D), lambda i: (0, i, 0)),
          pl.BlockSpec((ROW_BLK, 1), lambda i: (i, 0)),
          pl.BlockSpec((D, D), lambda i: (0, 0)),
          pl.BlockSpec((D, D), lambda i: (0, 0)),
          pl.BlockSpec((1, D), lambda i: (0, 0)),
          pl.BlockSpec((1, 1, ROW_BLK), lambda i: (i, 0, 0)),
          pl.BlockSpec((D, D), lambda i: (0, 0)),
          pl.BlockSpec((1, D), lambda i: (0, 0)),
          pl.BlockSpec((D, hd), lambda i: (0, 0)),
          pl.BlockSpec((1, hd), lambda i: (0, 0)),
          pl.BlockSpec((hd, 1), lambda i: (0, 0)),
          pl.BlockSpec((1, 1), lambda i: (0, 0)),
      ],
      out_specs=pl.BlockSpec((NUM_GRAPHS, 1), lambda i: (0, 0)),
      out_shape=jax.ShapeDtypeStruct((NUM_GRAPHS, 1), jnp.float32),
      scratch_shapes=[
          pltpu.VMEM((NUM_GRAPHS, D), jnp.float32),
          pltpu.VMEM((NUM_GRAPHS, 1), jnp.float32),
      ],
  )(h1, sp, invd, ws, wn, b1.reshape(1, D), batch.reshape(GRID, 1, ROW_BLK),
    wr0, br0.reshape(1, D), wr1, br1.reshape(1, hd), wr2, br2.reshape(1, 1))


@jax.jit
def kernel(x, edge_index, batch, W_self0, W_neigh0, b0, W_self1, W_neigh1, b1,
           Wr0, br0, Wr1, br1, Wr2, br2):
  src = edge_index[0].astype(jnp.int32)
  dst = edge_index[1].astype(jnp.int32)
  batch = batch.astype(jnp.int32)

  invd = _invdeg(dst)
  pad = jnp.arange(N_PAD_EDGES, dtype=jnp.int32) % 8
  srcp = jnp.concatenate([src, pad])
  dstp = jnp.concatenate([dst, N_NODES + pad])
  (s0p,) = _sc_scatter(srcp, dstp, x)
  h1 = _layer(x, s0p, invd, W_self0, W_neigh0, b0)
  (s1p,) = _sc_scatter(srcp, dstp, h1)
  return _readout(h1, s1p, invd, batch, W_self1, W_neigh1, b1,
                  Wr0, br0, Wr1, br1, Wr2, br2)


# speedup vs baseline: 1.2251x; 1.1610x over previous
import functools

import jax
import jax.numpy as jnp
from jax import lax
from jax.experimental import pallas as pl
from jax.experimental.pallas import tpu as pltpu
from jax.experimental.pallas import tpu_sc as plsc

N_NODES = 10000
N_EDGES = 320000
D = 128
NUM_GRAPHS = 512

NC, NS = 2, 16
NW = NC * NS
CHUNK = 112
NCHUNK = 2880
TILE_CHUNKS = NCHUNK // NW
N_PAD_EDGES = NCHUNK * CHUNK - N_EDGES
ACC_ROWS = N_NODES + 8
STRIPE = 624
TAIL = N_NODES - NS * STRIPE
DEGW = 16

ROW_BLK = 2000
GRID = N_NODES // ROW_BLK


def _make_sc_scatter():
  mesh = plsc.VectorSubcoreMesh(core_axis_name="c", subcore_axis_name="s")
  out_type = [jax.ShapeDtypeStruct((NC, N_NODES, D), jnp.float32)]
  scratch = [
      pltpu.VMEM((6, CHUNK), jnp.int32),
      pltpu.VMEM((6, CHUNK), jnp.int32),
      pltpu.VMEM((3, CHUNK, D), jnp.float32),
      pltpu.VMEM((24, D), jnp.float32),
      pltpu.VMEM_SHARED((ACC_ROWS, D), jnp.float32),
      pltpu.SemaphoreType.DMA,
      pltpu.SemaphoreType.DMA,
      pltpu.SemaphoreType.DMA,
      pltpu.SemaphoreType.DMA,
      pltpu.SemaphoreType.DMA,
      pltpu.SemaphoreType.DMA,
      pltpu.SemaphoreType.DMA,
      pltpu.SemaphoreType.DMA,
      pltpu.SemaphoreType.DMA,
      pltpu.SemaphoreType.DMA,
      pltpu.SemaphoreType.DMA,
      pltpu.SemaphoreType.DMA,
  ]

  def body(src_hbm, dst_hbm, y_hbm, out_hbm, sidx, didx, rows, zbuf, acc_sh,
           rsem0, rsem1, rsem2, ssem0, ssem1, ssem2,
           isem0, isem1, isem2, isem3, isem4, isem5):
    c = lax.axis_index("c")
    s = lax.axis_index("s")
    w = c * NS + s
    rsem = [rsem0, rsem1, rsem2]
    ssem = [ssem0, ssem1, ssem2]
    isem = [isem0, isem1, isem2, isem3, isem4, isem5]

    def iload(ch, j):
      pltpu.async_copy(src_hbm.at[pl.ds(ch * CHUNK, CHUNK)], sidx.at[j],
                       isem[j])
      pltpu.async_copy(dst_hbm.at[pl.ds(ch * CHUNK, CHUNK)], didx.at[j],
                       isem[j])

    def iwait(j):
      pltpu.make_async_copy(src_hbm.at[pl.ds(0, CHUNK)], sidx.at[j],
                            isem[j]).wait()
      pltpu.make_async_copy(dst_hbm.at[pl.ds(0, CHUNK)], didx.at[j],
                            isem[j]).wait()

    def gstart(p, j):
      pltpu.async_copy(y_hbm.at[sidx.at[j]], rows.at[p], rsem[p])

    def gwait(p):
      pltpu.make_async_copy(y_hbm.at[sidx.at[0]], rows.at[p],
                            rsem[p]).wait()

    def sstart(p, j):
      pltpu.async_copy(rows.at[p], acc_sh.at[didx.at[j]], ssem[p], add=True)

    def swait(p):
      pltpu.make_async_copy(rows.at[p], acc_sh.at[pl.ds(0, CHUNK)],
                            ssem[p]).wait()

    base = w * TILE_CHUNKS
    for j in range(5):
      iload(base + j, j)

    zv = jnp.zeros((16,), jnp.float32)
    for j in range(24):
      for k in range(D // 16):
        zbuf[j, pl.ds(k * 16, 16)] = zv
    zd = [pltpu.async_copy(zbuf, acc_sh.at[pl.ds(s * STRIPE + r * 24, 24)],
                           rsem0)
          for r in range(26)]

    @pl.when(s == NS - 1)
    def _():
      pltpu.sync_copy(zbuf.at[pl.ds(0, TAIL)],
                      acc_sh.at[pl.ds(NS * STRIPE, TAIL)])

    for d in zd:
      d.wait()

    iwait(0)
    gstart(0, 0)
    iwait(1)
    gstart(1, 1)
    plsc.subcore_barrier()

    def hexa(i, carry):
      c0 = i * 6
      for x in range(6):
        p, j = x % 3, x
        q, j2, j5 = (x + 2) % 3, (x + 2) % 6, (x + 5) % 6
        gwait(p)
        sstart(p, j)
        if x == 0:
          @pl.when(i > 0)
          def _(q=q):
            swait(q)
        else:
          swait(q)

        @pl.when(c0 + x + 2 < TILE_CHUNKS)
        def _(q=q, j2=j2):
          iwait(j2)
          gstart(q, j2)

        @pl.when(c0 + x + 5 < TILE_CHUNKS)
        def _(x=x, j5=j5, c0=c0):
          iload(base + c0 + x + 5, j5)

      return carry

    lax.fori_loop(0, TILE_CHUNKS // 6, hexa, 0)
    swait(2)
    plsc.subcore_barrier()

    pltpu.sync_copy(acc_sh.at[pl.ds(s * STRIPE, STRIPE)],
                    out_hbm.at[c, pl.ds(s * STRIPE, STRIPE)])

    @pl.when(s == NS - 1)
    def _():
      pltpu.sync_copy(acc_sh.at[pl.ds(NS * STRIPE, TAIL)],
                      out_hbm.at[c, pl.ds(NS * STRIPE, TAIL)])

  return pl.kernel(body, out_type=out_type, mesh=mesh, scratch_types=scratch)


_sc_scatter = _make_sc_scatter()


EDGE_BLK = 8000
EDGE_GRID = N_EDGES // EDGE_BLK
HI = (N_NODES + D - 1) // D


def _deg_body(dst_ref, inv_ref, acc):
  i = pl.program_id(0)

  @pl.when(i == 0)
  def _():
    acc[...] = jnp.zeros_like(acc)

  db = dst_ref[0]
  hi = db // D
  lo = db - hi * D
  ohi = (lax.broadcasted_iota(jnp.int32, (HI, EDGE_BLK), 0)
         == hi).astype(jnp.bfloat16)
  olo = (lax.broadcasted_iota(jnp.int32, (D, EDGE_BLK), 0)
         == lo).astype(jnp.bfloat16)
  acc[...] += lax.dot_general(ohi, olo, (((1,), (1,)), ((), ())),
                              preferred_element_type=jnp.float32)

  @pl.when(i == EDGE_GRID - 1)
  def _():
    inv_ref[...] = 1.0 / jnp.maximum(acc[...], 1.0)


def _invdeg(dst):
  out = pl.pallas_call(
      _deg_body,
      grid=(EDGE_GRID,),
      in_specs=[pl.BlockSpec((1, 1, EDGE_BLK), lambda i: (i, 0, 0))],
      out_specs=pl.BlockSpec((HI, D), lambda i: (0, 0)),
      out_shape=jax.ShapeDtypeStruct((HI, D), jnp.float32),
      scratch_shapes=[pltpu.VMEM((HI, D), jnp.float32)],
  )(dst.reshape(EDGE_GRID, 1, EDGE_BLK))
  return out.reshape(-1)[:N_NODES].reshape(N_NODES, 1)


def _layer_body(x_ref, sp_ref, inv_ref, ws_ref, wn_ref, b_ref, h_ref):
  agg = (sp_ref[0] + sp_ref[1]) * inv_ref[...]
  h_ref[...] = jnp.maximum(
      jnp.dot(x_ref[...], ws_ref[...], preferred_element_type=jnp.float32)
      + jnp.dot(agg, wn_ref[...], preferred_element_type=jnp.float32)
      + b_ref[...], 0.0)


def _layer(x, sp, invd, ws, wn, b):
  return pl.pallas_call(
      _layer_body,
      grid=(GRID,),
      in_specs=[
          pl.BlockSpec((ROW_BLK, D), lambda i: (i, 0)),
          pl.BlockSpec((NC, ROW_BLK, D), lambda i: (0, i, 0)),
          pl.BlockSpec((ROW_BLK, 1), lambda i: (i, 0)),
          pl.BlockSpec((D, D), lambda i: (0, 0)),
          pl.BlockSpec((D, D), lambda i: (0, 0)),
          pl.BlockSpec((1, D), lambda i: (0, 0)),
      ],
      out_specs=pl.BlockSpec((ROW_BLK, D), lambda i: (i, 0)),
      out_shape=jax.ShapeDtypeStruct((N_NODES, D), jnp.float32),
  )(x, sp, invd, ws, wn, b.reshape(1, D))


def _readout_body(h1_ref, sp_ref, inv_ref, ws_ref, wn_ref, b1_ref, b_ref,
                  wr0_ref, br0_ref, wr1_ref, br1_ref, wr2_ref, br2_ref,
                  out_ref, pooled, cnt):
  i = pl.program_id(0)

  @pl.when(i == 0)
  def _():
    pooled[...] = jnp.zeros_like(pooled)
    cnt[...] = jnp.zeros_like(cnt)

  agg = (sp_ref[0] + sp_ref[1]) * inv_ref[...]
  h = jnp.maximum(
      jnp.dot(h1_ref[...], ws_ref[...], preferred_element_type=jnp.float32)
      + jnp.dot(agg, wn_ref[...], preferred_element_type=jnp.float32)
      + b1_ref[...], 0.0)
  bb = b_ref[0]
  pt = (lax.broadcasted_iota(jnp.int32, (NUM_GRAPHS, ROW_BLK), 0)
        == bb).astype(jnp.float32)
  pooled[...] += jnp.dot(pt, h, preferred_element_type=jnp.float32,
                         precision=lax.Precision.HIGHEST)
  cnt[...] += jnp.sum(pt, axis=1, keepdims=True)

  @pl.when(i == GRID - 1)
  def _():
    pm = pooled[...] / jnp.maximum(cnt[...], 1.0)
    r = jnp.maximum(
        jnp.dot(pm, wr0_ref[...], preferred_element_type=jnp.float32)
        + br0_ref[...], 0.0)
    r = jnp.maximum(
        jnp.dot(r, wr1_ref[...], preferred_element_type=jnp.float32)
        + br1_ref[...], 0.0)
    out_ref[...] = jnp.dot(
        r, wr2_ref[...], preferred_element_type=jnp.float32) + br2_ref[...]


def _readout(h1, sp, invd, batch, ws, wn, b1, wr0, br0, wr1, br1, wr2, br2):
  hd = wr1.shape[1]
  return pl.pallas_call(
      _readout_body,
      grid=(GRID,),
      in_specs=[
          pl.BlockSpec((ROW_BLK, D), lambda i: (i, 0)),
          pl.BlockSpec((NC, ROW_BLK, D), lambda i: (0, i, 0)),
          pl.BlockSpec((ROW_BLK, 1), lambda i: (i, 0)),
          pl.BlockSpec((D, D), lambda i: (0, 0)),
          pl.BlockSpec((D, D), lambda i: (0, 0)),
          pl.BlockSpec((1, D), lambda i: (0, 0)),
          pl.BlockSpec((1, 1, ROW_BLK), lambda i: (i, 0, 0)),
          pl.BlockSpec((D, D), lambda i: (0, 0)),
          pl.BlockSpec((1, D), lambda i: (0, 0)),
          pl.BlockSpec((D, hd), lambda i: (0, 0)),
          pl.BlockSpec((1, hd), lambda i: (0, 0)),
          pl.BlockSpec((hd, 1), lambda i: (0, 0)),
          pl.BlockSpec((1, 1), lambda i: (0, 0)),
      ],
      out_specs=pl.BlockSpec((NUM_GRAPHS, 1), lambda i: (0, 0)),
      out_shape=jax.ShapeDtypeStruct((NUM_GRAPHS, 1), jnp.float32),
      scratch_shapes=[
          pltpu.VMEM((NUM_GRAPHS, D), jnp.float32),
          pltpu.VMEM((NUM_GRAPHS, 1), jnp.float32),
      ],
  )(h1, sp, invd, ws, wn, b1.reshape(1, D), batch.reshape(GRID, 1, ROW_BLK),
    wr0, br0.reshape(1, D), wr1, br1.reshape(1, hd), wr2, br2.reshape(1, 1))


@jax.jit
def kernel(x, edge_index, batch, W_self0, W_neigh0, b0, W_self1, W_neigh1, b1,
           Wr0, br0, Wr1, br1, Wr2, br2):
  src = edge_index[0].astype(jnp.int32)
  dst = edge_index[1].astype(jnp.int32)
  batch = batch.astype(jnp.int32)

  invd = _invdeg(dst)
  pad = jnp.arange(N_PAD_EDGES, dtype=jnp.int32) % 8
  srcp = jnp.concatenate([src, pad])
  dstp = jnp.concatenate([dst, N_NODES + pad])
  (s0p,) = _sc_scatter(srcp, dstp, x)
  h1 = _layer(x, s0p, invd, W_self0, W_neigh0, b0)
  (s1p,) = _sc_scatter(srcp, dstp, h1)
  return _readout(h1, s1p, invd, batch, W_self1, W_neigh1, b1,
                  Wr0, br0, Wr1, br1, Wr2, br2)
